# baseline (speedup 1.0000x reference)
import functools

import jax
import jax.numpy as jnp
import numpy as np
from jax import lax
from jax.experimental import pallas as pl
from jax.experimental.pallas import tpu as pltpu
from jax.experimental.pallas import tpu_sc as plsc

N = 10000
E = 320000
VOCAB = 1000
ENT = 16

NC = 2
NS = 16
NW = NC * NS
L = 16

EW = E // NW
CK1 = 80
CK2 = 80
NBUF = 4
NEB = 8

OWN = 640

NB = 80
NBLK = N // NB


def _ilv_perm(D):
    perm = np.zeros(D, dtype=np.int32)
    for g in range(D // 32):
        for j in range(16):
            perm[g * 32 + 2 * j] = g * 32 + j
            perm[g * 32 + 2 * j + 1] = g * 32 + 16 + j
    return perm


def _mesh():
    return plsc.VectorSubcoreMesh(core_axis_name="c", subcore_axis_name="s",
                                  num_cores=NC, num_subcores=NS)


def _emb_body(i0, i1, i2, i3, t0, t1, t2, t3, o0, o1, o2, o3,
              ibuf, gbuf, isem, gsem, osem):
    c = lax.axis_index("c")
    s = lax.axis_index("s")
    wid = c * NS + s
    idxs = (i0, i1, i2, i3)
    tabs = (t0, t1, t2, t3)
    outs = (o0, o1, o2, o3)
    for j in range(4):
        blk = wid + j * NW

        @pl.when(blk < NBLK)
        def _():
            r0 = blk * NB
            for t in range(4):
                pltpu.async_copy(idxs[t].at[pl.ds(r0, NB)], ibuf.at[t],
                                 isem)
            for t in range(4):
                pltpu.make_async_copy(idxs[t].at[pl.ds(r0, NB)],
                                      ibuf.at[t], isem).wait()
            for t in range(4):
                pltpu.async_copy(tabs[t].at[ibuf.at[t]], gbuf.at[t], gsem)
            for t in range(4):
                pltpu.make_async_copy(tabs[t].at[ibuf.at[t]], gbuf.at[t],
                                      gsem).wait()
            for t in range(4):
                pltpu.async_copy(gbuf.at[t], outs[t].at[pl.ds(r0, NB)],
                                 osem)
            for t in range(4):
                pltpu.make_async_copy(gbuf.at[t], outs[t].at[pl.ds(r0, NB)],
                                      osem).wait()


def _emb_gather(ds0, ds1, ss0, ss1, dt0, dt1, st0, st1):
    f = pl.kernel(
        _emb_body,
        out_type=tuple(jax.ShapeDtypeStruct((N, ENT), jnp.float32)
                       for _ in range(4)),
        mesh=_mesh(),
        compiler_params=pltpu.CompilerParams(use_tc_tiling_on_sc=False),
        scratch_types=[
            pltpu.VMEM((4, NB), jnp.int32),
            pltpu.VMEM((4, NB, ENT), jnp.float32),
            pltpu.SemaphoreType.DMA,
            pltpu.SemaphoreType.DMA,
            pltpu.SemaphoreType.DMA,
        ],
    )
    return f(ds0, ds1, ss0, ss1, dt0, dt1, st0, st1)


def _prop_body(D, ck, nchunk, x, edata, out, acc, ebuf,
               r0, r1, r2, r3, g0, g1, g2, g3, s0, s1, s2, s3,
               i0, i1, i2, i3, i4, i5, i6, i7):
    c = lax.axis_index("c")
    s = lax.axis_index("s")
    wid = c * NS + s
    NV = D // L
    rows = (r0, r1, r2, r3)
    gsem = (g0, g1, g2, g3)
    ssem = (s0, s1, s2, s3)
    isem = (i0, i1, i2, i3, i4, i5, i6, i7)

    def _z(i, _):
        for v in range(NV):
            r0[i, pl.ds(v * L, L)] = jnp.zeros((L,), jnp.float32)
        return _

    lax.fori_loop(0, 80, _z, None)
    ncp = jnp.where(s < NS - 1, OWN // 80, (N - (NS - 1) * OWN) // 80)

    def _zc(i, _):
        pltpu.sync_copy(r0.at[pl.ds(0, 80)], acc.at[pl.ds(s * OWN + i * 80, 80)])
        return _

    lax.fori_loop(0, ncp, _zc, None)

    erow = wid * nchunk
    pltpu.sync_copy(edata.at[erow], ebuf.at[0])
    for j in range(1, 4):
        pltpu.async_copy(edata.at[erow + j], ebuf.at[j], isem[j])
    pltpu.async_copy(x.at[ebuf.at[0, 0]], r0, g0)
    pltpu.make_async_copy(edata.at[erow], ebuf.at[1], isem[1]).wait()
    pltpu.async_copy(x.at[ebuf.at[1, 0]], r1, g1)
    plsc.subcore_barrier()

    def _body(k, b, eb):
        bw = (b + 2) % NBUF
        ew = (eb + 2) % NEB

        @pl.when(k >= 2)
        def _():
            pltpu.make_async_copy(rows[bw], acc.at[ebuf.at[ew, 1]],
                                  ssem[bw]).wait()

        @pl.when(k + 4 < nchunk)
        def _():
            pltpu.async_copy(edata.at[erow + k + 4], ebuf.at[(eb + 4) % NEB],
                             isem[(eb + 4) % NEB])

        @pl.when(k + 2 < nchunk)
        def _():
            pltpu.make_async_copy(edata.at[erow], ebuf.at[ew],
                                  isem[ew]).wait()
            pltpu.async_copy(x.at[ebuf.at[ew, 0]], rows[bw], gsem[bw])

        pltpu.make_async_copy(x.at[ebuf.at[eb, 0]], rows[b], gsem[b]).wait()

        def _scale(e, _):
            wspl = plsc.bitcast(
                plsc.load_gather(
                    ebuf, [jnp.full((L,), eb, jnp.int32),
                           jnp.full((L,), 2, jnp.int32),
                           jnp.full((L,), e, jnp.int32)]), jnp.float32)
            for v in range(NV):
                sl = pl.ds(v * L, L)
                rows[b][e, sl] = rows[b][e, sl] * wspl
            return _

        lax.fori_loop(0, ck, _scale, None, unroll=5)
        pltpu.async_copy(rows[b], acc.at[ebuf.at[eb, 1]], ssem[b], add=True)

    def _oct(q, _):
        for j in range(NEB):
            _body(q * NEB + j, j % NBUF, j)
        return _

    lax.fori_loop(0, nchunk // NEB, _oct, None)
    for k in range(nchunk - nchunk % NEB, nchunk):
        _body(k, k % NBUF, k % NEB)
    for k in (nchunk - 2, nchunk - 1):
        b, eb = k % NBUF, k % NEB
        pltpu.make_async_copy(rows[b], acc.at[ebuf.at[eb, 1]],
                              ssem[b]).wait()
    plsc.subcore_barrier()

    def _wc(i, _):
        pltpu.sync_copy(acc.at[pl.ds(s * OWN + i * 80, 80)],
                        out.at[pl.ds(c * N + s * OWN + i * 80, 80)])
        return _

    lax.fori_loop(0, ncp, _wc, None)


def _prop(x, edata, D, ck):
    nchunk = EW // ck
    f = pl.kernel(
        functools.partial(_prop_body, D, ck, nchunk),
        out_type=jax.ShapeDtypeStruct((2 * N, D), jnp.float32),
        mesh=_mesh(),
        compiler_params=pltpu.CompilerParams(use_tc_tiling_on_sc=False,
                                             needs_layout_passes=False),
        scratch_types=(
            [pltpu.VMEM_SHARED((N, D), jnp.float32),
             pltpu.VMEM((NEB, 3, ck), jnp.int32)]
            + [pltpu.VMEM((ck, D), jnp.float32)] * NBUF
            + [pltpu.SemaphoreType.DMA] * (2 * NBUF + NEB)
        ),
    )
    return f(x, edata)


def _prop1_body(D, ck, nchunk, t4, ix4, xd, edata, out, x1, acc, ebuf, ibuf4,
                r0, r1, r2, r3, g0, g1, g2, g3, s0, s1, s2, s3,
                i0, i1, i2, i3, i4, i5, i6, i7):
    c = lax.axis_index("c")
    s = lax.axis_index("s")
    wid = c * NS + s
    NV = D // L
    rows = (r0, r1, r2, r3)
    gsem = (g0, g1, g2, g3)
    ssem = (s0, s1, s2, s3)
    isem = (i0, i1, i2, i3, i4, i5, i6, i7)
    ncp = jnp.where(s < NS - 1, OWN // 80, (N - (NS - 1) * OWN) // 80)

    def _p0(i, _):
        r = s * OWN + i * 80
        pltpu.sync_copy(xd.at[pl.ds(r, 80)], r0)
        for t in range(4):
            pltpu.sync_copy(ix4.at[t, pl.ds(r, 80)], ibuf4.at[t])
        for t in range(3):
            pltpu.async_copy(t4.at[ibuf4.at[t]], rows[t + 1], gsem[t])
        for t in range(3):
            pltpu.make_async_copy(t4.at[ibuf4.at[t]], rows[t + 1],
                                  gsem[t]).wait()

        def _add3(e, _2):
            for v in range(NV):
                sl = pl.ds(v * L, L)
                r0[e, sl] = (r0[e, sl] + r1[e, sl]) + (r2[e, sl] + r3[e, sl])
            return _2

        lax.fori_loop(0, 80, _add3, None, unroll=4)
        pltpu.async_copy(t4.at[ibuf4.at[3]], r1, gsem[3])
        pltpu.make_async_copy(t4.at[ibuf4.at[3]], r1, gsem[3]).wait()

        def _add1(e, _2):
            for v in range(NV):
                sl = pl.ds(v * L, L)
                r0[e, sl] = r0[e, sl] + r1[e, sl]
            return _2

        lax.fori_loop(0, 80, _add1, None, unroll=4)
        pltpu.sync_copy(r0, x1.at[pl.ds(r, 80)])
        return _

    lax.fori_loop(0, ncp, _p0, None)

    def _z(i, _):
        for v in range(NV):
            r0[i, pl.ds(v * L, L)] = jnp.zeros((L,), jnp.float32)
        return _

    lax.fori_loop(0, 80, _z, None)

    def _zc(i, _):
        pltpu.sync_copy(r0.at[pl.ds(0, 80)],
                        acc.at[pl.ds(s * OWN + i * 80, 80)])
        return _

    lax.fori_loop(0, ncp, _zc, None)

    erow = wid * nchunk
    pltpu.sync_copy(edata.at[erow], ebuf.at[0])
    for j in range(1, 4):
        pltpu.async_copy(edata.at[erow + j], ebuf.at[j], isem[j])
    plsc.subcore_barrier()
    pltpu.async_copy(x1.at[ebuf.at[0, 0]], r0, g0)
    pltpu.make_async_copy(edata.at[erow], ebuf.at[1], isem[1]).wait()
    pltpu.async_copy(x1.at[ebuf.at[1, 0]], r1, g1)

    def _body(k, b, eb):
        bw = (b + 2) % NBUF
        ew = (eb + 2) % NEB

        @pl.when(k >= 2)
        def _():
            pltpu.make_async_copy(rows[bw], acc.at[ebuf.at[ew, 1]],
                                  ssem[bw]).wait()

        @pl.when(k + 4 < nchunk)
        def _():
            pltpu.async_copy(edata.at[erow + k + 4], ebuf.at[(eb + 4) % NEB],
                             isem[(eb + 4) % NEB])

        @pl.when(k + 2 < nchunk)
        def _():
            pltpu.make_async_copy(edata.at[erow], ebuf.at[ew],
                                  isem[ew]).wait()
            pltpu.async_copy(x1.at[ebuf.at[ew, 0]], rows[bw], gsem[bw])

        pltpu.make_async_copy(x1.at[ebuf.at[eb, 0]], rows[b], gsem[b]).wait()

        def _scale(e, _):
            wspl = plsc.bitcast(
                plsc.load_gather(
                    ebuf, [jnp.full((L,), eb, jnp.int32),
                           jnp.full((L,), 2, jnp.int32),
                           jnp.full((L,), e, jnp.int32)]), jnp.float32)
            for v in range(NV):
                sl = pl.ds(v * L, L)
                rows[b][e, sl] = rows[b][e, sl] * wspl
            return _

        lax.fori_loop(0, ck, _scale, None, unroll=5)
        pltpu.async_copy(rows[b], acc.at[ebuf.at[eb, 1]], ssem[b], add=True)

    def _oct(q, _):
        for j in range(NEB):
            _body(q * NEB + j, j % NBUF, j)
        return _

    lax.fori_loop(0, nchunk // NEB, _oct, None)
    for k in range(nchunk - nchunk % NEB, nchunk):
        _body(k, k % NBUF, k % NEB)
    for k in (nchunk - 2, nchunk - 1):
        b, eb = k % NBUF, k % NEB
        pltpu.make_async_copy(rows[b], acc.at[ebuf.at[eb, 1]],
                              ssem[b]).wait()
    plsc.subcore_barrier()

    def _wc(i, _):
        pltpu.sync_copy(acc.at[pl.ds(s * OWN + i * 80, 80)],
                        out.at[pl.ds(c * N + s * OWN + i * 80, 80)])
        return _

    lax.fori_loop(0, ncp, _wc, None)


def _prop1(t4, ix4, xd, edata):
    ck = CK1
    nchunk = EW // ck
    f = pl.kernel(
        functools.partial(_prop1_body, 128, ck, nchunk),
        out_type=(jax.ShapeDtypeStruct((2 * N, 128), jnp.float32),
                  jax.ShapeDtypeStruct((N, 128), jnp.float32)),
        mesh=_mesh(),
        compiler_params=pltpu.CompilerParams(use_tc_tiling_on_sc=False,
                                             needs_layout_passes=False),
        scratch_types=(
            [pltpu.VMEM_SHARED((N, 128), jnp.float32),
             pltpu.VMEM((NEB, 3, ck), jnp.int32),
             pltpu.VMEM((4, 80), jnp.int32)]
            + [pltpu.VMEM((ck, 128), jnp.float32)] * NBUF
            + [pltpu.SemaphoreType.DMA] * (2 * NBUF + NEB)
        ),
    )
    return f(t4, ix4, xd, edata)


RB = 1000


def _tmix_body(dd, sd, w1, tabs, oxd, ot):
    oxd[...] = (
        jnp.dot(dd[...], w1[32:64, :], preferred_element_type=jnp.float32)
        + jnp.dot(sd[...], w1[96:128, :], preferred_element_type=jnp.float32))
    starts = (0, 16, 64, 80)
    for t in range(4):
        ot[t] = jnp.dot(tabs[t], w1[starts[t]:starts[t] + 16, :],
                        preferred_element_type=jnp.float32)


def _tmix(ddx, sdx, w1, tabs):
    return pl.pallas_call(
        _tmix_body,
        out_shape=(jax.ShapeDtypeStruct((N, 128), jnp.float32),
                   jax.ShapeDtypeStruct((4, VOCAB, 128), jnp.float32)),
    )(ddx, sdx, w1, tabs)


def _mix_body(g0, g1, dd, g2, g3, sd, w1, o):
    x = jnp.dot(g0[...], w1[0:16, :], preferred_element_type=jnp.float32)
    x += jnp.dot(g1[...], w1[16:32, :], preferred_element_type=jnp.float32)
    x += jnp.dot(dd[...], w1[32:64, :], preferred_element_type=jnp.float32)
    x += jnp.dot(g2[...], w1[64:80, :], preferred_element_type=jnp.float32)
    x += jnp.dot(g3[...], w1[80:96, :], preferred_element_type=jnp.float32)
    x += jnp.dot(sd[...], w1[96:128, :], preferred_element_type=jnp.float32)
    o[...] = x


def _mix(g0, g1, dd, g2, g3, sd, w1):
    grid = (N // RB,)
    bs_e = pl.BlockSpec((RB, ENT), lambda i: (i, 0))
    bs_d = pl.BlockSpec((RB, 32), lambda i: (i, 0))
    bs_w = pl.BlockSpec((128, 128), lambda i: (0, 0))
    return pl.pallas_call(
        _mix_body,
        grid=grid,
        in_specs=[bs_e, bs_e, bs_d, bs_e, bs_e, bs_d, bs_w],
        out_specs=pl.BlockSpec((RB, 128), lambda i: (i, 0)),
        out_shape=jax.ShapeDtypeStruct((N, 128), jnp.float32),
    )(g0, g1, dd, g2, g3, sd, w1)


def _act_body(pa, pb, b1, w2, o):
    h = jnp.maximum(pa[...] + pb[...] + b1[...], 0.0)
    o[...] = jnp.dot(h, w2[...], preferred_element_type=jnp.float32)


def _act(p1, b1, w2):
    grid = (N // RB,)
    return pl.pallas_call(
        _act_body,
        grid=grid,
        in_specs=[
            pl.BlockSpec((RB, 128), lambda i: (i, 0)),
            pl.BlockSpec((RB, 128), lambda i: (i + N // RB, 0)),
            pl.BlockSpec((1, 128), lambda i: (0, 0)),
            pl.BlockSpec((128, 64), lambda i: (0, 0)),
        ],
        out_specs=pl.BlockSpec((RB, 64), lambda i: (i, 0)),
        out_shape=jax.ShapeDtypeStruct((N, 64), jnp.float32),
    )(p1, p1, b1.reshape(1, 128), w2)


def _fin_body(pa, pb, b2, o):
    o[...] = pa[...] + pb[...] + b2[...]


def _fin(p2, b2):
    grid = (N // RB,)
    return pl.pallas_call(
        _fin_body,
        grid=grid,
        in_specs=[
            pl.BlockSpec((RB, 64), lambda i: (i, 0)),
            pl.BlockSpec((RB, 64), lambda i: (i + N // RB, 0)),
            pl.BlockSpec((1, 64), lambda i: (0, 0)),
        ],
        out_specs=pl.BlockSpec((RB, 64), lambda i: (i, 0)),
        out_shape=jax.ShapeDtypeStruct((N, 64), jnp.float32),
    )(p2, p2, b2.reshape(1, 64))


def kernel(static_dense_x, static_sparse_x, dynamic_dense_x, dynamic_sparse_x,
           edges, weights, static_emb_0, static_emb_1, dyn_emb_0, dyn_emb_1,
           W1, b1, W2, b2):
    ss0 = static_sparse_x[:, 0].astype(jnp.int32)
    ss1 = static_sparse_x[:, 1].astype(jnp.int32)
    ds0 = dynamic_sparse_x[0, :, 0].astype(jnp.int32)
    ds1 = dynamic_sparse_x[0, :, 1].astype(jnp.int32)
    wbits = lax.bitcast_convert_type(weights[0], jnp.int32)

    def _edata(ck):
        nchunk = EW // ck
        return jnp.stack([edges[0, 0].astype(jnp.int32).reshape(-1, ck),
                          edges[0, 1].astype(jnp.int32).reshape(-1, ck),
                          wbits.reshape(-1, ck)], axis=1)

    edata1 = _edata(CK1)
    edata2 = edata1
    ddx = dynamic_dense_x[0]

    tabs = jnp.stack([dyn_emb_0, dyn_emb_1, static_emb_0, static_emb_1])
    ix4 = jnp.stack([ds0, ds1 + VOCAB, ss0 + 2 * VOCAB, ss1 + 3 * VOCAB])

    xd, tt = _tmix(ddx, static_dense_x, W1, tabs)
    p1, _ = _prop1(tt.reshape(4 * VOCAB, 128), ix4, xd, edata1)
    g = _act(p1, b1, W2)
    p2 = _prop(g, edata2, 64, CK2)
    return _fin(p2, b2)


# speedup vs baseline: 1.2255x; 1.2255x over previous
import functools

import jax
import jax.numpy as jnp
import numpy as np
from jax import lax
from jax.experimental import pallas as pl
from jax.experimental.pallas import tpu as pltpu
from jax.experimental.pallas import tpu_sc as plsc

N = 10000
E = 320000
VOCAB = 1000
ENT = 16

NC = 2
NS = 16
NW = NC * NS
L = 16

EW = E // NW
CK1 = 80
CK2 = 80
NBUF = 4
NEB = 8

OWN = 640

NB = 80
NBLK = N // NB


def _ilv_perm(D):
    perm = np.zeros(D, dtype=np.int32)
    for g in range(D // 32):
        for j in range(16):
            perm[g * 32 + 2 * j] = g * 32 + j
            perm[g * 32 + 2 * j + 1] = g * 32 + 16 + j
    return perm


def _mesh():
    return plsc.VectorSubcoreMesh(core_axis_name="c", subcore_axis_name="s",
                                  num_cores=NC, num_subcores=NS)


def _emb_body(i0, i1, i2, i3, t0, t1, t2, t3, o0, o1, o2, o3,
              ibuf, gbuf, isem, gsem, osem):
    c = lax.axis_index("c")
    s = lax.axis_index("s")
    wid = c * NS + s
    idxs = (i0, i1, i2, i3)
    tabs = (t0, t1, t2, t3)
    outs = (o0, o1, o2, o3)
    for j in range(4):
        blk = wid + j * NW

        @pl.when(blk < NBLK)
        def _():
            r0 = blk * NB
            for t in range(4):
                pltpu.async_copy(idxs[t].at[pl.ds(r0, NB)], ibuf.at[t],
                                 isem)
            for t in range(4):
                pltpu.make_async_copy(idxs[t].at[pl.ds(r0, NB)],
                                      ibuf.at[t], isem).wait()
            for t in range(4):
                pltpu.async_copy(tabs[t].at[ibuf.at[t]], gbuf.at[t], gsem)
            for t in range(4):
                pltpu.make_async_copy(tabs[t].at[ibuf.at[t]], gbuf.at[t],
                                      gsem).wait()
            for t in range(4):
                pltpu.async_copy(gbuf.at[t], outs[t].at[pl.ds(r0, NB)],
                                 osem)
            for t in range(4):
                pltpu.make_async_copy(gbuf.at[t], outs[t].at[pl.ds(r0, NB)],
                                      osem).wait()


def _emb_gather(ds0, ds1, ss0, ss1, dt0, dt1, st0, st1):
    f = pl.kernel(
        _emb_body,
        out_type=tuple(jax.ShapeDtypeStruct((N, ENT), jnp.float32)
                       for _ in range(4)),
        mesh=_mesh(),
        compiler_params=pltpu.CompilerParams(use_tc_tiling_on_sc=False),
        scratch_types=[
            pltpu.VMEM((4, NB), jnp.int32),
            pltpu.VMEM((4, NB, ENT), jnp.float32),
            pltpu.SemaphoreType.DMA,
            pltpu.SemaphoreType.DMA,
            pltpu.SemaphoreType.DMA,
        ],
    )
    return f(ds0, ds1, ss0, ss1, dt0, dt1, st0, st1)


def _prop_body(D, ck, nchunk, x, edata, out, acc, ebuf,
               r0, r1, r2, r3, g0, g1, g2, g3, s0, s1, s2, s3,
               i0, i1, i2, i3, i4, i5, i6, i7):
    c = lax.axis_index("c")
    s = lax.axis_index("s")
    wid = c * NS + s
    NV = D // L
    rows = (r0, r1, r2, r3)
    gsem = (g0, g1, g2, g3)
    ssem = (s0, s1, s2, s3)
    isem = (i0, i1, i2, i3, i4, i5, i6, i7)

    def _z(i, _):
        for v in range(NV):
            r0[i, pl.ds(v * L, L)] = jnp.zeros((L,), jnp.float32)
        return _

    lax.fori_loop(0, 80, _z, None)
    ncp = jnp.where(s < NS - 1, OWN // 80, (N - (NS - 1) * OWN) // 80)

    def _zc(i, _):
        pltpu.sync_copy(r0.at[pl.ds(0, 80)], acc.at[pl.ds(s * OWN + i * 80, 80)])
        return _

    lax.fori_loop(0, ncp, _zc, None)

    erow = wid * nchunk
    pltpu.sync_copy(edata.at[erow], ebuf.at[0])
    for j in range(1, 4):
        pltpu.async_copy(edata.at[erow + j], ebuf.at[j], isem[j])
    pltpu.async_copy(x.at[ebuf.at[0, 0]], r0, g0)
    pltpu.make_async_copy(edata.at[erow], ebuf.at[1], isem[1]).wait()
    pltpu.async_copy(x.at[ebuf.at[1, 0]], r1, g1)
    plsc.subcore_barrier()

    def _body(k, b, eb):
        bw = (b + 2) % NBUF
        ew = (eb + 2) % NEB

        @pl.when(k >= 2)
        def _():
            pltpu.make_async_copy(rows[bw], acc.at[ebuf.at[ew, 1]],
                                  ssem[bw]).wait()

        @pl.when(k + 4 < nchunk)
        def _():
            pltpu.async_copy(edata.at[erow + k + 4], ebuf.at[(eb + 4) % NEB],
                             isem[(eb + 4) % NEB])

        @pl.when(k + 2 < nchunk)
        def _():
            pltpu.make_async_copy(edata.at[erow], ebuf.at[ew],
                                  isem[ew]).wait()
            pltpu.async_copy(x.at[ebuf.at[ew, 0]], rows[bw], gsem[bw])

        pltpu.make_async_copy(x.at[ebuf.at[eb, 0]], rows[b], gsem[b]).wait()

        def _scale(e, _):
            wspl = plsc.bitcast(
                plsc.load_gather(
                    ebuf, [jnp.full((L,), eb, jnp.int32),
                           jnp.full((L,), 2, jnp.int32),
                           jnp.full((L,), e, jnp.int32)]), jnp.float32)
            for v in range(NV):
                sl = pl.ds(v * L, L)
                rows[b][e, sl] = rows[b][e, sl] * wspl
            return _

        lax.fori_loop(0, ck, _scale, None, unroll=5)
        pltpu.async_copy(rows[b], acc.at[ebuf.at[eb, 1]], ssem[b], add=True)

    def _oct(q, _):
        for j in range(NEB):
            _body(q * NEB + j, j % NBUF, j)
        return _

    lax.fori_loop(0, nchunk // NEB, _oct, None)
    for k in range(nchunk - nchunk % NEB, nchunk):
        _body(k, k % NBUF, k % NEB)
    for k in (nchunk - 2, nchunk - 1):
        b, eb = k % NBUF, k % NEB
        pltpu.make_async_copy(rows[b], acc.at[ebuf.at[eb, 1]],
                              ssem[b]).wait()
    plsc.subcore_barrier()

    def _wc(i, _):
        pltpu.sync_copy(acc.at[pl.ds(s * OWN + i * 80, 80)],
                        out.at[pl.ds(c * N + s * OWN + i * 80, 80)])
        return _

    lax.fori_loop(0, ncp, _wc, None)


def _prop(x, edata, D, ck):
    nchunk = EW // ck
    f = pl.kernel(
        functools.partial(_prop_body, D, ck, nchunk),
        out_type=jax.ShapeDtypeStruct((2 * N, D), jnp.float32),
        mesh=_mesh(),
        compiler_params=pltpu.CompilerParams(use_tc_tiling_on_sc=False,
                                             needs_layout_passes=False),
        scratch_types=(
            [pltpu.VMEM_SHARED((N, D), jnp.float32),
             pltpu.VMEM((NEB, 3, ck), jnp.int32)]
            + [pltpu.VMEM((ck, D), jnp.float32)] * NBUF
            + [pltpu.SemaphoreType.DMA] * (2 * NBUF + NEB)
        ),
    )
    return f(x, edata)


RB = 1000


def _mix_body(g0, g1, dd, g2, g3, sd, w1, o):
    x = jnp.dot(g0[...], w1[0:16, :], preferred_element_type=jnp.float32)
    x += jnp.dot(g1[...], w1[16:32, :], preferred_element_type=jnp.float32)
    x += jnp.dot(dd[...], w1[32:64, :], preferred_element_type=jnp.float32)
    x += jnp.dot(g2[...], w1[64:80, :], preferred_element_type=jnp.float32)
    x += jnp.dot(g3[...], w1[80:96, :], preferred_element_type=jnp.float32)
    x += jnp.dot(sd[...], w1[96:128, :], preferred_element_type=jnp.float32)
    o[...] = x


def _mix(g0, g1, dd, g2, g3, sd, w1):
    grid = (N // RB,)
    bs_e = pl.BlockSpec((RB, ENT), lambda i: (i, 0))
    bs_d = pl.BlockSpec((RB, 32), lambda i: (i, 0))
    bs_w = pl.BlockSpec((128, 128), lambda i: (0, 0))
    return pl.pallas_call(
        _mix_body,
        grid=grid,
        in_specs=[bs_e, bs_e, bs_d, bs_e, bs_e, bs_d, bs_w],
        out_specs=pl.BlockSpec((RB, 128), lambda i: (i, 0)),
        out_shape=jax.ShapeDtypeStruct((N, 128), jnp.float32),
    )(g0, g1, dd, g2, g3, sd, w1)


def _act_body(pa, pb, b1, w2, o):
    h = jnp.maximum(pa[...] + pb[...] + b1[...], 0.0)
    o[...] = jnp.dot(h, w2[...], preferred_element_type=jnp.float32)


def _act(p1, b1, w2):
    grid = (N // RB,)
    return pl.pallas_call(
        _act_body,
        grid=grid,
        in_specs=[
            pl.BlockSpec((RB, 128), lambda i: (i, 0)),
            pl.BlockSpec((RB, 128), lambda i: (i + N // RB, 0)),
            pl.BlockSpec((1, 128), lambda i: (0, 0)),
            pl.BlockSpec((128, 64), lambda i: (0, 0)),
        ],
        out_specs=pl.BlockSpec((RB, 64), lambda i: (i, 0)),
        out_shape=jax.ShapeDtypeStruct((N, 64), jnp.float32),
    )(p1, p1, b1.reshape(1, 128), w2)


def _fin_body(pa, pb, b2, o):
    o[...] = pa[...] + pb[...] + b2[...]


def _fin(p2, b2):
    grid = (N // RB,)
    return pl.pallas_call(
        _fin_body,
        grid=grid,
        in_specs=[
            pl.BlockSpec((RB, 64), lambda i: (i, 0)),
            pl.BlockSpec((RB, 64), lambda i: (i + N // RB, 0)),
            pl.BlockSpec((1, 64), lambda i: (0, 0)),
        ],
        out_specs=pl.BlockSpec((RB, 64), lambda i: (i, 0)),
        out_shape=jax.ShapeDtypeStruct((N, 64), jnp.float32),
    )(p2, p2, b2.reshape(1, 64))


def kernel(static_dense_x, static_sparse_x, dynamic_dense_x, dynamic_sparse_x,
           edges, weights, static_emb_0, static_emb_1, dyn_emb_0, dyn_emb_1,
           W1, b1, W2, b2):
    ss0 = static_sparse_x[:, 0].astype(jnp.int32)
    ss1 = static_sparse_x[:, 1].astype(jnp.int32)
    ds0 = dynamic_sparse_x[0, :, 0].astype(jnp.int32)
    ds1 = dynamic_sparse_x[0, :, 1].astype(jnp.int32)
    wbits = lax.bitcast_convert_type(weights[0], jnp.int32)

    def _edata(ck):
        nchunk = EW // ck
        return jnp.stack([edges[0, 0].astype(jnp.int32).reshape(-1, ck),
                          edges[0, 1].astype(jnp.int32).reshape(-1, ck),
                          wbits.reshape(-1, ck)], axis=1)

    edata1 = _edata(CK1)
    edata2 = edata1
    ddx = dynamic_dense_x[0]

    g0, g1, g2, g3 = _emb_gather(ds0, ds1, ss0, ss1,
                                 dyn_emb_0, dyn_emb_1,
                                 static_emb_0, static_emb_1)
    x1 = _mix(g0, g1, ddx, g2, g3, static_dense_x, W1)
    p1 = _prop(x1, edata1, 128, CK1)
    g = _act(p1, b1, W2)
    p2 = _prop(g, edata2, 64, CK2)
    return _fin(p2, b2)
